# baseline (speedup 1.0000x reference)
import functools

import jax
import jax.numpy as jnp
import numpy as np
from jax.experimental import pallas as pl
from jax.experimental.pallas import tpu as pltpu

N_NODES = 10000
HID = 64
_NFULL = N_NODES // 128
_NTAIL = N_NODES - _NFULL * 128
_INV_N = float(np.float32(1e-4))
_EPS = float(np.float32(1e-5))


def _sum_nodes(h):
    acc = h[:, 0:128]
    for j in range(1, _NFULL):
        acc = acc + h[:, 128 * j:128 * (j + 1)]
    tail = h[:, _NFULL * 128:N_NODES]
    acc = jnp.concatenate([acc[:, 0:_NTAIL] + tail, acc[:, _NTAIL:]], axis=1)
    g = acc[:, 0:8]
    for k in range(1, 16):
        g = g + acc[:, 8 * k:8 * k + 8]
    g = g[:, 0:4] + g[:, 4:8]
    g = g[:, 0:2] + g[:, 2:4]
    return g[:, 0:1] + g[:, 1:2]


def _mlp_body(depth, with_cls):
    def body(hin_ref, w0t_ref, wts_ref, b_ref, g_ref, be_ref, *rest):
        if with_cls:
            cwt_ref, cb_ref, out_ref = rest
        else:
            (out_ref,) = rest
        h = jnp.transpose(hin_ref[...])
        for i in range(depth):
            w = w0t_ref[...] if i == 0 else wts_ref[i - 1]
            h = jnp.dot(w, h, preferred_element_type=jnp.float32)
            h = h + b_ref[i]
            h = jnp.maximum(h, 0.0)
            mu = _sum_nodes(h) * _INV_N
            d = h - mu
            vs = _sum_nodes(d * d)
            s = jnp.sqrt(vs * _INV_N + _EPS)
            h = g_ref[i] * d / s + be_ref[i]
        h = jnp.maximum(h, 0.0)
        if with_cls:
            h = jnp.dot(cwt_ref[...], h, preferred_element_type=jnp.float32)
            h = h + cb_ref[...]
        out_ref[...] = jnp.transpose(h)

    return body


def _mlp_call(hin, params, cls=None):
    depth = len(params)
    w0t = params[0][0].T
    wts = jnp.stack([params[i][0].T for i in range(1, depth)])
    bs = jnp.stack([p[1] for p in params])[:, :, None]
    gs = jnp.stack([p[2] for p in params])[:, :, None]
    bes = jnp.stack([p[3] for p in params])[:, :, None]
    args = [hin, w0t, wts, bs, gs, bes]
    if cls is not None:
        args += [cls[0].T, cls[1].reshape(-1, 1)]
    out_ch = cls[0].shape[1] if cls is not None else params[-1][0].shape[1]
    return pl.pallas_call(
        _mlp_body(depth, cls is not None),
        out_shape=jax.ShapeDtypeStruct((hin.shape[0], out_ch), jnp.float32),
        compiler_params=pltpu.CompilerParams(
            vmem_limit_bytes=120 * 1024 * 1024,
        ),
    )(*args)


def kernel(x, edge_attr, edge_index, conv_params, cls_W, cls_b):
    del edge_attr
    src = edge_index[0]
    dst = edge_index[1]

    agg1 = jnp.zeros_like(x).at[dst].add(x[src])
    h1 = _mlp_call(x + agg1, conv_params[0])
    agg2 = jnp.zeros_like(h1).at[dst].add(h1[src])
    return _mlp_call(h1 + agg2, conv_params[1], cls=(cls_W, cls_b))
